# baseline (speedup 1.0000x reference)
import functools

import jax
import jax.numpy as jnp
from jax import lax
from jax.experimental import pallas as pl
from jax.experimental.pallas import tpu as pltpu
from jax.experimental.pallas import tpu_sc as plsc

SLICE = 64
L = 16


def _sc_slice_gather(n_piece, row_off, d, rows_per_w, chunk_rows, nc):
    n_chunks = rows_per_w // chunk_rows
    assert n_chunks % 2 == 0
    groups = chunk_rows // L
    j_steps = SLICE // L

    mesh = plsc.VectorSubcoreMesh(core_axis_name="c", subcore_axis_name="s")

    @functools.partial(
        pl.kernel,
        mesh=mesh,
        compiler_params=pltpu.CompilerParams(needs_layout_passes=False),
        out_type=jax.ShapeDtypeStruct((n_piece, SLICE), jnp.float32),
        scratch_types=[
            pltpu.VMEM((chunk_rows, d), jnp.float32),
            pltpu.VMEM((chunk_rows, d), jnp.float32),
            pltpu.VMEM((chunk_rows, SLICE), jnp.float32),
            pltpu.VMEM((chunk_rows, SLICE), jnp.float32),
            pltpu.VMEM((chunk_rows,), jnp.int32),
            pltpu.VMEM((chunk_rows,), jnp.int32),
            pltpu.SemaphoreType.DMA,
            pltpu.SemaphoreType.DMA,
            pltpu.SemaphoreType.DMA,
            pltpu.SemaphoreType.DMA,
        ],
    )
    def k(in_hbm, idx_hbm, out_hbm, in_v0, in_v1, out_v0, out_v1,
          idx_v0, idx_v1, sem_in0, sem_in1, sem_out0, sem_out1):
        in_v = (in_v0, in_v1)
        out_v = (out_v0, out_v1)
        idx_v = (idx_v0, idx_v1)
        sem_in = (sem_in0, sem_in1)
        sem_out = (sem_out0, sem_out1)
        wid = lax.axis_index("s") * nc + lax.axis_index("c")
        base_row = wid * rows_per_w
        iota = lax.iota(jnp.int32, L)

        def in_copy(c, b):
            row0 = row_off + base_row + c * chunk_rows
            return (
                pltpu.make_async_copy(
                    in_hbm.at[pl.ds(row0, chunk_rows)], in_v[b], sem_in[b]),
                pltpu.make_async_copy(
                    idx_hbm.at[pl.ds(row0, chunk_rows)], idx_v[b], sem_in[b]),
            )

        def out_copy(c, b):
            row0 = base_row + c * chunk_rows
            return pltpu.make_async_copy(
                out_v[b], out_hbm.at[pl.ds(row0, chunk_rows)], sem_out[b])

        def compute(b):
            @plsc.parallel_loop(0, groups, 1)
            def group_body(g):
                svec = idx_v[b][pl.ds(g * L, L)]
                for r in range(L):
                    row = g * L + r
                    s_b = jnp.take_along_axis(
                        svec, jnp.full((L,), r, jnp.int32), axis=0)
                    rvec = jnp.full((L,), row, jnp.int32)
                    col0 = s_b + iota
                    for j in range(j_steps):
                        vals = plsc.load_gather(
                            in_v[b], [rvec, col0 + (j * L)])
                        out_v[b][row, pl.ds(j * L, L)] = vals

        for b in range(2):
            for cp in in_copy(b, b):
                cp.start()

        def pair_body(i, carry):
            for b in range(2):
                c = i * 2 + b
                for cp in in_copy(c, b):
                    cp.wait()

                @pl.when(i > 0)
                def _():
                    out_copy(c, b).wait()

                compute(b)
                out_copy(c, b).start()

                @pl.when(c + 2 < n_chunks)
                def _():
                    for cp in in_copy(c + 2, b):
                        cp.start()
            return carry

        lax.fori_loop(0, n_chunks // 2, pair_body, 0)
        for b in range(2):
            out_copy(n_chunks - 2 + b, b).wait()

    return k


def kernel(input_tensor, slices_index, slice_len):
    n, d = input_tensor.shape
    adj_idx = slices_index.astype(jnp.int32) + (
        jnp.asarray(slice_len, jnp.int32) - SLICE)

    num_workers = 32
    nc = 2
    chunk_rows = 128
    pieces_n = 4
    n_piece = n // pieces_n
    rows_per_w = n_piece // num_workers
    outs = []
    for t in range(pieces_n):
        f = _sc_slice_gather(n_piece, t * n_piece, d, rows_per_w,
                             chunk_rows, nc)
        outs.append(f(input_tensor, adj_idx))
    return jnp.concatenate(outs, axis=0)
